# baseline (speedup 1.0000x reference)
import functools

import jax
import jax.numpy as jnp
from jax import lax
from jax.experimental import pallas as pl
from jax.experimental.pallas import tpu as pltpu
from jax.experimental.pallas import tpu_sc as plsc

L = 16
ROWS = 2
NBUF = 3
UNROLL = 2


def _sc_pe_add(B, S, D):
    NC, NS = 2, 16
    NW = NC * NS
    sw = S // NW
    n_chunks = sw // ROWS
    R = B * ROWS

    mesh = plsc.VectorSubcoreMesh(core_axis_name="c", subcore_axis_name="s")

    @functools.partial(
        pl.kernel,
        out_type=jax.ShapeDtypeStruct((B, S, D), jnp.float32),
        mesh=mesh,
        scratch_types=(
            [pltpu.VMEM((R, D), jnp.float32) for _ in range(NBUF)]
            + [pltpu.VMEM((ROWS, D), jnp.float32) for _ in range(NBUF)]
            + [pltpu.SemaphoreType.DMA for _ in range(3 * NBUF)]
        ),
    )
    def body(x_hbm, pe_hbm, out_hbm, *scratch):
        xa = scratch[:NBUF]
        pe_v = scratch[NBUF:2 * NBUF]
        sem_ld = scratch[2 * NBUF:2 * NBUF + NBUF]
        sem_pe = scratch[2 * NBUF + NBUF:2 * NBUF + 2 * NBUF]
        sem_st = scratch[2 * NBUF + 2 * NBUF:]

        wid = lax.axis_index("s") * NC + lax.axis_index("c")
        base = wid * sw

        ld_desc = [None] * n_chunks
        pe_desc = [None] * n_chunks
        st_desc = [None] * n_chunks

        def issue_loads(c):
            j = c % NBUF
            s0 = base + c * ROWS
            pe_desc[c] = pltpu.async_copy(
                pe_hbm.at[pl.ds(s0, ROWS)], pe_v[j], sem_pe[j]
            )
            ld_desc[c] = [
                pltpu.async_copy(
                    x_hbm.at[b, pl.ds(s0, ROWS)],
                    xa[j].at[pl.ds(b * ROWS, ROWS)],
                    sem_ld[j],
                )
                for b in range(B)
            ]

        def issue_stores(c):
            j = c % NBUF
            s0 = base + c * ROWS
            st_desc[c] = [
                pltpu.async_copy(
                    xa[j].at[pl.ds(b * ROWS, ROWS)],
                    out_hbm.at[b, pl.ds(s0, ROWS)],
                    sem_st[j],
                )
                for b in range(B)
            ]

        issue_loads(0)
        if n_chunks > 1:
            issue_loads(1)

        for c in range(n_chunks):
            j = c % NBUF
            pe_desc[c].wait()
            for d in ld_desc[c]:
                d.wait()

            def cbody(i, carry, j=j):
                for r in range(R):
                    for u in range(UNROLL):
                        off = i * (L * UNROLL) + u * L
                        xa[j][r, pl.ds(off, L)] = (
                            xa[j][r, pl.ds(off, L)]
                            + pe_v[j][r % ROWS, pl.ds(off, L)]
                        )
                return carry
            lax.fori_loop(0, D // (L * UNROLL), cbody, 0)

            issue_stores(c)
            if c >= 1:
                for d in st_desc[c - 1]:
                    d.wait()
            if c + 2 < n_chunks:
                issue_loads(c + 2)

        for d in st_desc[n_chunks - 1]:
            d.wait()

    return body


def kernel(x, pe_table):
    B, S, D = x.shape
    fn = _sc_pe_add(B, S, D)
    return fn(x, pe_table)


# speedup vs baseline: 2.7157x; 2.7157x over previous
import functools

import jax
import jax.numpy as jnp
from jax import lax
from jax.experimental import pallas as pl
from jax.experimental.pallas import tpu as pltpu
from jax.experimental.pallas import tpu_sc as plsc

L = 16
UNROLL = 8


def _sc_pe_add(B, S, D):
    NC, NS = 2, 16
    NW = NC * NS
    sw = S // NW
    K = sw // 2

    mesh = plsc.VectorSubcoreMesh(core_axis_name="c", subcore_axis_name="s")

    @functools.partial(
        pl.kernel,
        out_type=jax.ShapeDtypeStruct((B, S, D), jnp.float32),
        mesh=mesh,
        scratch_types=(
            [pltpu.VMEM((B, D), jnp.float32) for _ in range(2)]
            + [pltpu.VMEM((B, D), jnp.float32) for _ in range(2)]
            + [pltpu.VMEM((1, D), jnp.float32) for _ in range(2)]
            + [pltpu.SemaphoreType.DMA for _ in range(6)]
        ),
    )
    def body(x_hbm, pe_hbm, out_hbm, *scratch):
        xa = scratch[0:2]
        ov = scratch[2:4]
        pe_v = scratch[4:6]
        sem_ld = scratch[6:8]
        sem_pe = scratch[8:10]
        sem_st = scratch[10:12]

        wid = lax.axis_index("s") * NC + lax.axis_index("c")
        base = wid * sw

        def issue_loads(jj, s0):
            pltpu.async_copy(pe_hbm.at[pl.ds(s0, 1)], pe_v[jj], sem_pe[jj])
            for b in range(B):
                pltpu.async_copy(
                    x_hbm.at[b, pl.ds(s0, 1)],
                    xa[jj].at[pl.ds(b, 1)],
                    sem_ld[jj],
                )

        issue_loads(0, base)
        issue_loads(1, base + 1)

        def step(k, carry):
            for jj in range(2):
                c = 2 * k + jj
                s0 = base + c
                pltpu.make_async_copy(
                    pe_hbm.at[pl.ds(s0, 1)], pe_v[jj], sem_pe[jj]
                ).wait()
                for b in range(B):
                    pltpu.make_async_copy(
                        x_hbm.at[b, pl.ds(s0, 1)],
                        xa[jj].at[pl.ds(b, 1)],
                        sem_ld[jj],
                    ).wait()

                @pl.when(k > 0)
                def _(jj=jj, s0=s0):
                    for b in range(B):
                        pltpu.make_async_copy(
                            ov[jj].at[pl.ds(b, 1)],
                            out_hbm.at[b, pl.ds(s0, 1)],
                            sem_st[jj],
                        ).wait()

                @plsc.parallel_loop(0, D // L, unroll=UNROLL)
                def cbody(i, jj=jj):
                    off = i * L
                    p = pe_v[jj][0, pl.ds(off, L)]
                    for b in range(B):
                        ov[jj][b, pl.ds(off, L)] = xa[jj][b, pl.ds(off, L)] + p

                for b in range(B):
                    pltpu.async_copy(
                        ov[jj].at[pl.ds(b, 1)],
                        out_hbm.at[b, pl.ds(s0, 1)],
                        sem_st[jj],
                    )

                @pl.when(k < K - 1)
                def _(jj=jj, s0=s0):
                    issue_loads(jj, s0 + 2)
            return carry

        lax.fori_loop(0, K, step, 0)

        for jj in range(2):
            s0 = base + sw - 2 + jj
            for b in range(B):
                pltpu.make_async_copy(
                    ov[jj].at[pl.ds(b, 1)],
                    out_hbm.at[b, pl.ds(s0, 1)],
                    sem_st[jj],
                ).wait()

    return body


def kernel(x, pe_table):
    B, S, D = x.shape
    fn = _sc_pe_add(B, S, D)
    return fn(x, pe_table)
